# Initial kernel scaffold; baseline (speedup 1.0000x reference)
import jax
import jax.numpy as jnp
from jax.experimental import pallas as pl


def kernel(x, edge_index, adj_low_w, adj_low_unnorm_w, W_low1, W_high1, W_mlp1, att_low1, att_high1, att_mlp1, att_vec1, W_low2, W_high2, W_mlp2, att_low2, att_high2, att_mlp2, att_vec2):
    raise NotImplementedError("write your pallas kernel here")


# speedup vs baseline: 4.7611x; 4.7611x over previous
import functools

import jax
import jax.numpy as jnp
from jax import lax
from jax.experimental import pallas as pl
from jax.experimental.pallas import tpu as pltpu
from jax.experimental.pallas import tpu_sc as plsc

N = 10000
E = 320000
NS = 16
C = 80
PER_TILE = E // NS
NCHUNK = PER_TILE // C
ACCN = 10240
ROWS_PER_TILE = ACCN // NS
ZB = 80


def _make_spmm(D):
  mesh = plsc.VectorSubcoreMesh(core_axis_name="c", subcore_axis_name="s")

  @functools.partial(
      pl.kernel,
      out_type=jax.ShapeDtypeStruct((2 * ACCN, D), jnp.float32),
      mesh=mesh,
      scratch_types=[
          pltpu.VMEM((C,), jnp.int32),
          pltpu.VMEM((C,), jnp.int32),
          pltpu.VMEM((C,), jnp.float32),
          pltpu.VMEM((C, D), jnp.float32),
          pltpu.VMEM((ZB, D), jnp.float32),
          pltpu.VMEM_SHARED((ACCN, D), jnp.float32),
          pltpu.SemaphoreType.DMA,
      ],
  )
  def spmm(table_hbm, src_hbm, dst_hbm, w_hbm, out_hbm,
           src_v, dst_v, w_v, rows_v, zbuf, acc_sh, sem):
    c = lax.axis_index("c")
    s = lax.axis_index("s")

    zvec = jnp.zeros((16,), jnp.float32)

    @pl.loop(0, ZB)
    def _fill(i):
      for k in range(D // 16):
        zbuf[i, pl.ds(k * 16, 16)] = zvec

    row0 = s * ROWS_PER_TILE

    @pl.loop(0, ROWS_PER_TILE // ZB)
    def _zero(z):
      pltpu.sync_copy(zbuf, acc_sh.at[pl.ds(row0 + z * ZB, ZB)])

    plsc.subcore_barrier()

    ebase = s * PER_TILE
    cN = c * N

    @pl.loop(0, NCHUNK)
    def _chunk(j):
      off = pl.multiple_of(ebase + j * C, 8)
      pltpu.sync_copy(src_hbm.at[pl.ds(off, C)], src_v)
      pltpu.sync_copy(dst_hbm.at[pl.ds(off, C)], dst_v)
      pltpu.sync_copy(w_hbm.at[pl.ds(off, C)], w_v)
      for g in range(C // 16):
        sl = pl.ds(g * 16, 16)
        src_v[sl] = src_v[sl] + cN
      pltpu.async_copy(table_hbm.at[src_v], rows_v, sem).wait()

      for g in range(C // 16):
        wv = w_v[pl.ds(g * 16, 16)]
        for i in range(16):
          wi = wv[i]
          row = g * 16 + i
          for k in range(D // 16):
            sl = pl.ds(k * 16, 16)
            rows_v[row, sl] = rows_v[row, sl] * wi

      pltpu.sync_copy(rows_v, acc_sh.at[dst_v], add=True)

    plsc.subcore_barrier()

    pltpu.sync_copy(acc_sh.at[pl.ds(row0, ROWS_PER_TILE)],
                    out_hbm.at[pl.ds(c * ACCN + row0, ROWS_PER_TILE)])

  return spmm


_spmm128 = _make_spmm(128)


def _make_spmm_esplit():
  D = 128
  per_tile = E // 32
  nchunk = per_tile // C
  mesh = plsc.VectorSubcoreMesh(core_axis_name="c", subcore_axis_name="s")

  @functools.partial(
      pl.kernel,
      out_type=jax.ShapeDtypeStruct((2 * ACCN, D), jnp.float32),
      mesh=mesh,
      scratch_types=[
          pltpu.VMEM((C,), jnp.int32),
          pltpu.VMEM((C,), jnp.int32),
          pltpu.VMEM((C,), jnp.float32),
          pltpu.VMEM((C, D), jnp.float32),
          pltpu.VMEM((ZB, D), jnp.float32),
          pltpu.VMEM_SHARED((ACCN, D), jnp.float32),
          pltpu.SemaphoreType.DMA,
      ],
  )
  def spmm(table_hbm, src_hbm, dst_hbm, w_hbm, out_hbm,
           src_v, dst_v, w_v, rows_v, zbuf, acc_sh, sem):
    c = lax.axis_index("c")
    s = lax.axis_index("s")

    zvec = jnp.zeros((16,), jnp.float32)

    @pl.loop(0, ZB)
    def _fill(i):
      for k in range(D // 16):
        zbuf[i, pl.ds(k * 16, 16)] = zvec

    row0 = s * ROWS_PER_TILE

    @pl.loop(0, ROWS_PER_TILE // ZB)
    def _zero(z):
      pltpu.sync_copy(zbuf, acc_sh.at[pl.ds(row0 + z * ZB, ZB)])

    plsc.subcore_barrier()

    ebase = (c * NS + s) * per_tile

    @pl.loop(0, nchunk)
    def _chunk(j):
      off = pl.multiple_of(ebase + j * C, 8)
      pltpu.sync_copy(src_hbm.at[pl.ds(off, C)], src_v)
      pltpu.sync_copy(dst_hbm.at[pl.ds(off, C)], dst_v)
      pltpu.sync_copy(w_hbm.at[pl.ds(off, C)], w_v)
      pltpu.async_copy(table_hbm.at[src_v], rows_v, sem).wait()

      for g in range(C // 16):
        wv = w_v[pl.ds(g * 16, 16)]
        for i in range(16):
          wi = wv[i]
          row = g * 16 + i
          for k in range(D // 16):
            sl = pl.ds(k * 16, 16)
            rows_v[row, sl] = rows_v[row, sl] * wi

      pltpu.sync_copy(rows_v, acc_sh.at[dst_v], add=True)

    plsc.subcore_barrier()

    pltpu.sync_copy(acc_sh.at[pl.ds(row0, ROWS_PER_TILE)],
                    out_hbm.at[pl.ds(c * ACCN + row0, ROWS_PER_TILE)])

  return spmm


_spmm2 = _make_spmm_esplit()


def _stackmm(h, wstack):
  n, k = h.shape
  d = wstack.shape[2]
  r = 2000
  nb = n // r

  def body(h_ref, w_ref, o_ref):
    t = pl.program_id(0)
    acc = jnp.dot(h_ref[...], w_ref[0], preferred_element_type=jnp.float32)
    o_ref[...] = jnp.where(t == 2, jnp.maximum(acc, 0.0), acc)

  return pl.pallas_call(
      body,
      grid=(3, nb),
      in_specs=[
          pl.BlockSpec((r, k), lambda t, b: (b, 0)),
          pl.BlockSpec((1, k, d), lambda t, b: (t, 0, 0)),
      ],
      out_specs=pl.BlockSpec((r, d), lambda t, b: (t * nb + b, 0)),
      out_shape=jax.ShapeDtypeStruct((3 * n, d), jnp.float32),
  )(h, wstack)


def _l2mm(h, wcat, wm):
  n, k = h.shape
  r = 2000
  nb = n // r

  def body(h_ref, wc_ref, wm_ref, t2_ref, hm_ref):
    hv = h_ref[...]
    t2_ref[...] = jnp.dot(hv, wc_ref[...], preferred_element_type=jnp.float32)
    hm_ref[...] = jnp.maximum(
        jnp.dot(hv, wm_ref[...], preferred_element_type=jnp.float32), 0.0)

  return pl.pallas_call(
      body,
      grid=(nb,),
      in_specs=[
          pl.BlockSpec((r, k), lambda b: (b, 0)),
          pl.BlockSpec((k, 128), lambda b: (0, 0)),
          pl.BlockSpec((k, 64), lambda b: (0, 0)),
      ],
      out_specs=[
          pl.BlockSpec((r, 128), lambda b: (b, 0)),
          pl.BlockSpec((r, 64), lambda b: (b, 0)),
      ],
      out_shape=[
          jax.ShapeDtypeStruct((n, 128), jnp.float32),
          jax.ShapeDtypeStruct((n, 64), jnp.float32),
      ],
  )(h, wcat, wm)


def _att_combine(ol, oh, om, al_ref, ah_ref, am_ref, av_ref, final_relu):
  g0 = jax.nn.sigmoid(jnp.sum(ol * al_ref[...], axis=1, keepdims=True))
  g1 = jax.nn.sigmoid(jnp.sum(oh * ah_ref[...], axis=1, keepdims=True))
  g2 = jax.nn.sigmoid(jnp.sum(om * am_ref[...], axis=1, keepdims=True))
  third = 1.0 / 3.0
  t0 = (g0 * av_ref[0, 0] + g1 * av_ref[1, 0] + g2 * av_ref[2, 0]) * third
  t1 = (g0 * av_ref[0, 1] + g1 * av_ref[1, 1] + g2 * av_ref[2, 1]) * third
  t2 = (g0 * av_ref[0, 2] + g1 * av_ref[1, 2] + g2 * av_ref[2, 2]) * third
  m = jnp.maximum(jnp.maximum(t0, t1), t2)
  e0 = jnp.exp(t0 - m)
  e1 = jnp.exp(t1 - m)
  e2 = jnp.exp(t2 - m)
  inv = 1.0 / (e0 + e1 + e2)
  out = 3.0 * ((e0 * inv) * ol + (e1 * inv) * oh + (e2 * inv) * om)
  if final_relu:
    out = jnp.maximum(out, 0.0)
  return out


def _attention(s_low, s_high, stack, al_t, ah_t, am_t, av, final_relu):
  d = s_low.shape[1]
  r = 1000
  nb = N // r

  def body(sl_ref, sh_ref, hh_ref, hm_ref, al_ref, ah_ref, am_ref, av_ref,
           o_ref):
    ol = jnp.maximum(sl_ref[...], 0.0)
    oh = jnp.maximum(hh_ref[...] - sh_ref[...], 0.0)
    om = hm_ref[...]
    o_ref[...] = _att_combine(ol, oh, om, al_ref, ah_ref, am_ref, av_ref,
                              final_relu)

  return pl.pallas_call(
      body,
      grid=(nb,),
      in_specs=[
          pl.BlockSpec((r, d), lambda b: (b, 0)),
          pl.BlockSpec((r, d), lambda b: (b, 0)),
          pl.BlockSpec((r, d), lambda b, _nb=nb: (b + _nb, 0)),
          pl.BlockSpec((r, d), lambda b, _nb=nb: (b + 2 * _nb, 0)),
          pl.BlockSpec((1, d), lambda b: (0, 0)),
          pl.BlockSpec((1, d), lambda b: (0, 0)),
          pl.BlockSpec((1, d), lambda b: (0, 0)),
          pl.BlockSpec(memory_space=pltpu.SMEM),
      ],
      out_specs=pl.BlockSpec((r, d), lambda b: (b, 0)),
      out_shape=jax.ShapeDtypeStruct((N, d), jnp.float32),
  )(s_low, s_high, stack, stack, al_t, ah_t, am_t, av)


def _attention2(part0, part1, table2, hm2, al_t, ah_t, am_t, av):
  r = 1000
  nb = N // r

  def body(a_ref, b_ref, t2_ref, hm_ref, al_ref, ah_ref, am_ref, av_ref,
           o_ref):
    ssum = a_ref[...] + b_ref[...]
    hh = t2_ref[...][:, 64:128]
    ol = jnp.maximum(ssum[:, 0:64], 0.0)
    oh = jnp.maximum(hh - ssum[:, 64:128], 0.0)
    om = hm_ref[...]
    o_ref[...] = _att_combine(ol, oh, om, al_ref, ah_ref, am_ref, av_ref,
                              False)

  return pl.pallas_call(
      body,
      grid=(nb,),
      in_specs=[
          pl.BlockSpec((r, 128), lambda b: (b, 0)),
          pl.BlockSpec((r, 128), lambda b: (b, 0)),
          pl.BlockSpec((r, 128), lambda b: (b, 0)),
          pl.BlockSpec((r, 64), lambda b: (b, 0)),
          pl.BlockSpec((1, 64), lambda b: (0, 0)),
          pl.BlockSpec((1, 64), lambda b: (0, 0)),
          pl.BlockSpec((1, 64), lambda b: (0, 0)),
          pl.BlockSpec(memory_space=pltpu.SMEM),
      ],
      out_specs=pl.BlockSpec((r, 64), lambda b: (b, 0)),
      out_shape=jax.ShapeDtypeStruct((N, 64), jnp.float32),
  )(part0, part1, table2, hm2, al_t, ah_t, am_t, av)


def kernel(x, edge_index, adj_low_w, adj_low_unnorm_w, W_low1, W_high1,
           W_mlp1, att_low1, att_high1, att_mlp1, att_vec1, W_low2, W_high2,
           W_mlp2, att_low2, att_high2, att_mlp2, att_vec2):
  src = edge_index[0]
  dst = edge_index[1]

  w1 = jnp.stack([W_low1, W_high1, W_mlp1])
  stack1 = _stackmm(x, w1)
  sc1 = _spmm128(stack1, src, dst, adj_low_w)
  fea1 = _attention(sc1[:N], sc1[ACCN:ACCN + N], stack1,
                    att_low1.reshape(1, -1), att_high1.reshape(1, -1),
                    att_mlp1.reshape(1, -1), att_vec1, final_relu=True)

  wcat2 = jnp.concatenate([W_low2, W_high2], axis=1)
  table2, hm2 = _l2mm(fea1, wcat2, W_mlp2)
  sc2 = _spmm2(table2, src, dst, adj_low_w)
  fea2 = _attention2(sc2[:N], sc2[ACCN:ACCN + N], table2, hm2,
                     att_low2.reshape(1, -1), att_high2.reshape(1, -1),
                     att_mlp2.reshape(1, -1), att_vec2)
  return fea2


# speedup vs baseline: 10.8786x; 2.2849x over previous
import functools

import jax
import jax.numpy as jnp
from jax import lax
from jax.experimental import pallas as pl
from jax.experimental.pallas import tpu as pltpu
from jax.experimental.pallas import tpu_sc as plsc

N = 10000
E = 320000
NS = 16
C = 80
PER_TILE = E // NS
NCHUNK = PER_TILE // C
ACCN = 10240
ROWS_PER_TILE = ACCN // NS
ZB = 80


def _make_spmm(D, edge_split):
  ntiles = 32 if edge_split else NS
  per_tile = E // ntiles
  nchunk = per_tile // C
  npairs = nchunk // 2
  mesh = plsc.VectorSubcoreMesh(core_axis_name="c", subcore_axis_name="s")

  @functools.partial(
      pl.kernel,
      out_type=jax.ShapeDtypeStruct((2 * ACCN, D), jnp.float32),
      mesh=mesh,
      scratch_types=[
          pltpu.VMEM((2, C), jnp.int32),
          pltpu.VMEM((2, C), jnp.int32),
          pltpu.VMEM((2, C), jnp.float32),
          pltpu.VMEM((C, D), jnp.float32),
          pltpu.VMEM((C, D), jnp.float32),
          pltpu.VMEM((ZB, D), jnp.float32),
          pltpu.VMEM_SHARED((ACCN, D), jnp.float32),
          pltpu.SemaphoreType.DMA,
          pltpu.SemaphoreType.DMA,
          pltpu.SemaphoreType.DMA,
          pltpu.SemaphoreType.DMA,
          pltpu.SemaphoreType.DMA,
          pltpu.SemaphoreType.DMA,
          pltpu.SemaphoreType.DMA,
          pltpu.SemaphoreType.DMA,
      ],
  )
  def spmm(table_hbm, src_hbm, dst_hbm, w_hbm, out_hbm,
           src_v, dst_v, w_v, rows0, rows1, zbuf, acc_sh,
           semi0, semi1, semd0, semd1, semg0, semg1, sems0, sems1):
    c = lax.axis_index("c")
    s = lax.axis_index("s")
    tid = c * NS + s if edge_split else s
    ebase = tid * per_tile
    cN = c * N
    rows = (rows0, rows1)
    semi = (semi0, semi1)
    semd = (semd0, semd1)
    semg = (semg0, semg1)
    sems = (sems0, sems1)

    def fetch_src(j, b):
      off = pl.multiple_of(ebase + j * C, 8)
      pltpu.async_copy(src_hbm.at[pl.ds(off, C)], src_v.at[b], semi[b])

    def wait_src(j, b):
      off = pl.multiple_of(ebase + j * C, 8)
      pltpu.make_async_copy(src_hbm.at[pl.ds(off, C)], src_v.at[b],
                            semi[b]).wait()

    def fetch_dw(j, b):
      off = pl.multiple_of(ebase + j * C, 8)
      pltpu.async_copy(dst_hbm.at[pl.ds(off, C)], dst_v.at[b], semd[b])
      pltpu.async_copy(w_hbm.at[pl.ds(off, C)], w_v.at[b], semd[b])

    def wait_dw(j, b):
      off = pl.multiple_of(ebase + j * C, 8)
      pltpu.make_async_copy(dst_hbm.at[pl.ds(off, C)], dst_v.at[b],
                            semd[b]).wait()
      pltpu.make_async_copy(w_hbm.at[pl.ds(off, C)], w_v.at[b],
                            semd[b]).wait()

    def gather(b):
      if not edge_split:
        for g in range(C // 16):
          sl = pl.ds(g * 16, 16)
          src_v[b, sl] = src_v[b, sl] + cN
      pltpu.async_copy(table_hbm.at[src_v.at[b]], rows[b], semg[b])

    def wait_gather(b):
      pltpu.make_async_copy(table_hbm.at[src_v.at[b]], rows[b],
                            semg[b]).wait()

    def scale(b):
      for g in range(C // 16):
        wv = w_v[b, pl.ds(g * 16, 16)]
        for i in range(16):
          wi = wv[i]
          row = g * 16 + i
          for k in range(D // 16):
            sl = pl.ds(k * 16, 16)
            rows[b][row, sl] = rows[b][row, sl] * wi

    def scatter(b):
      pltpu.async_copy(rows[b], acc_sh.at[dst_v.at[b]], sems[b], add=True)

    def wait_scatter(b):
      pltpu.make_async_copy(rows[b], acc_sh.at[dst_v.at[b]], sems[b]).wait()

    zvec = jnp.zeros((16,), jnp.float32)

    @pl.loop(0, ZB)
    def _fill(i):
      for k in range(D // 16):
        zbuf[i, pl.ds(k * 16, 16)] = zvec

    row0 = s * ROWS_PER_TILE

    @pl.loop(0, ROWS_PER_TILE // ZB)
    def _zero(z):
      pltpu.sync_copy(zbuf, acc_sh.at[pl.ds(row0 + z * ZB, ZB)])

    fetch_src(0, 0)
    fetch_dw(0, 0)
    fetch_src(1, 1)
    fetch_dw(1, 1)
    wait_src(0, 0)
    gather(0)
    wait_src(1, 1)
    gather(1)
    plsc.subcore_barrier()

    @pl.loop(0, npairs)
    def _pair(p):
      a = 2 * p
      for (j, b) in ((a, 0), (a + 1, 1)):
        wait_dw(j, b)
        wait_gather(b)

        @pl.when(j + 2 < nchunk)
        def _pref_src():
          fetch_src(j + 2, b)

        scale(b)
        scatter(b)
      for (j, b) in ((a, 0), (a + 1, 1)):
        wait_scatter(b)

        @pl.when(j + 2 < nchunk)
        def _next():
          fetch_dw(j + 2, b)
          wait_src(j + 2, b)
          gather(b)

    if nchunk % 2:
      j = nchunk - 1
      wait_dw(j, 0)
      wait_gather(0)
      scale(0)
      pltpu.sync_copy(rows0, acc_sh.at[dst_v.at[0]], add=True)

    plsc.subcore_barrier()

    pltpu.sync_copy(acc_sh.at[pl.ds(row0, ROWS_PER_TILE)],
                    out_hbm.at[pl.ds(c * ACCN + row0, ROWS_PER_TILE)])

  return spmm


_spmm128 = _make_spmm(128, edge_split=False)
_spmm2 = _make_spmm(128, edge_split=True)


def _stackmm(h, wstack):
  n, k = h.shape
  d = wstack.shape[2]
  r = 2000
  nb = n // r

  def body(h_ref, w_ref, o_ref):
    t = pl.program_id(0)
    acc = jnp.dot(h_ref[...], w_ref[0], preferred_element_type=jnp.float32)
    o_ref[...] = jnp.where(t == 2, jnp.maximum(acc, 0.0), acc)

  return pl.pallas_call(
      body,
      grid=(3, nb),
      in_specs=[
          pl.BlockSpec((r, k), lambda t, b: (b, 0)),
          pl.BlockSpec((1, k, d), lambda t, b: (t, 0, 0)),
      ],
      out_specs=pl.BlockSpec((r, d), lambda t, b: (t * nb + b, 0)),
      out_shape=jax.ShapeDtypeStruct((3 * n, d), jnp.float32),
  )(h, wstack)


def _l2mm(h, wcat, wm):
  n, k = h.shape
  r = 2000
  nb = n // r

  def body(h_ref, wc_ref, wm_ref, t2_ref, hm_ref):
    hv = h_ref[...]
    t2_ref[...] = jnp.dot(hv, wc_ref[...], preferred_element_type=jnp.float32)
    hm_ref[...] = jnp.maximum(
        jnp.dot(hv, wm_ref[...], preferred_element_type=jnp.float32), 0.0)

  return pl.pallas_call(
      body,
      grid=(nb,),
      in_specs=[
          pl.BlockSpec((r, k), lambda b: (b, 0)),
          pl.BlockSpec((k, 128), lambda b: (0, 0)),
          pl.BlockSpec((k, 64), lambda b: (0, 0)),
      ],
      out_specs=[
          pl.BlockSpec((r, 128), lambda b: (b, 0)),
          pl.BlockSpec((r, 64), lambda b: (b, 0)),
      ],
      out_shape=[
          jax.ShapeDtypeStruct((n, 128), jnp.float32),
          jax.ShapeDtypeStruct((n, 64), jnp.float32),
      ],
  )(h, wcat, wm)


def _att_combine(ol, oh, om, al_ref, ah_ref, am_ref, av_ref, final_relu):
  g0 = jax.nn.sigmoid(jnp.sum(ol * al_ref[...], axis=1, keepdims=True))
  g1 = jax.nn.sigmoid(jnp.sum(oh * ah_ref[...], axis=1, keepdims=True))
  g2 = jax.nn.sigmoid(jnp.sum(om * am_ref[...], axis=1, keepdims=True))
  third = 1.0 / 3.0
  t0 = (g0 * av_ref[0, 0] + g1 * av_ref[1, 0] + g2 * av_ref[2, 0]) * third
  t1 = (g0 * av_ref[0, 1] + g1 * av_ref[1, 1] + g2 * av_ref[2, 1]) * third
  t2 = (g0 * av_ref[0, 2] + g1 * av_ref[1, 2] + g2 * av_ref[2, 2]) * third
  m = jnp.maximum(jnp.maximum(t0, t1), t2)
  e0 = jnp.exp(t0 - m)
  e1 = jnp.exp(t1 - m)
  e2 = jnp.exp(t2 - m)
  inv = 1.0 / (e0 + e1 + e2)
  out = 3.0 * ((e0 * inv) * ol + (e1 * inv) * oh + (e2 * inv) * om)
  if final_relu:
    out = jnp.maximum(out, 0.0)
  return out


def _attention(s_low, s_high, stack, al_t, ah_t, am_t, av, final_relu):
  d = s_low.shape[1]
  r = 1000
  nb = N // r

  def body(sl_ref, sh_ref, hh_ref, hm_ref, al_ref, ah_ref, am_ref, av_ref,
           o_ref):
    ol = jnp.maximum(sl_ref[...], 0.0)
    oh = jnp.maximum(hh_ref[...] - sh_ref[...], 0.0)
    om = hm_ref[...]
    o_ref[...] = _att_combine(ol, oh, om, al_ref, ah_ref, am_ref, av_ref,
                              final_relu)

  return pl.pallas_call(
      body,
      grid=(nb,),
      in_specs=[
          pl.BlockSpec((r, d), lambda b: (b, 0)),
          pl.BlockSpec((r, d), lambda b: (b, 0)),
          pl.BlockSpec((r, d), lambda b, _nb=nb: (b + _nb, 0)),
          pl.BlockSpec((r, d), lambda b, _nb=nb: (b + 2 * _nb, 0)),
          pl.BlockSpec((1, d), lambda b: (0, 0)),
          pl.BlockSpec((1, d), lambda b: (0, 0)),
          pl.BlockSpec((1, d), lambda b: (0, 0)),
          pl.BlockSpec(memory_space=pltpu.SMEM),
      ],
      out_specs=pl.BlockSpec((r, d), lambda b: (b, 0)),
      out_shape=jax.ShapeDtypeStruct((N, d), jnp.float32),
  )(s_low, s_high, stack, stack, al_t, ah_t, am_t, av)


def _attention2(part0, part1, table2, hm2, al_t, ah_t, am_t, av):
  r = 1000
  nb = N // r

  def body(a_ref, b_ref, t2_ref, hm_ref, al_ref, ah_ref, am_ref, av_ref,
           o_ref):
    ssum = a_ref[...] + b_ref[...]
    hh = t2_ref[...][:, 64:128]
    ol = jnp.maximum(ssum[:, 0:64], 0.0)
    oh = jnp.maximum(hh - ssum[:, 64:128], 0.0)
    om = hm_ref[...]
    o_ref[...] = _att_combine(ol, oh, om, al_ref, ah_ref, am_ref, av_ref,
                              False)

  return pl.pallas_call(
      body,
      grid=(nb,),
      in_specs=[
          pl.BlockSpec((r, 128), lambda b: (b, 0)),
          pl.BlockSpec((r, 128), lambda b: (b, 0)),
          pl.BlockSpec((r, 128), lambda b: (b, 0)),
          pl.BlockSpec((r, 64), lambda b: (b, 0)),
          pl.BlockSpec((1, 64), lambda b: (0, 0)),
          pl.BlockSpec((1, 64), lambda b: (0, 0)),
          pl.BlockSpec((1, 64), lambda b: (0, 0)),
          pl.BlockSpec(memory_space=pltpu.SMEM),
      ],
      out_specs=pl.BlockSpec((r, 64), lambda b: (b, 0)),
      out_shape=jax.ShapeDtypeStruct((N, 64), jnp.float32),
  )(part0, part1, table2, hm2, al_t, ah_t, am_t, av)


def kernel(x, edge_index, adj_low_w, adj_low_unnorm_w, W_low1, W_high1,
           W_mlp1, att_low1, att_high1, att_mlp1, att_vec1, W_low2, W_high2,
           W_mlp2, att_low2, att_high2, att_mlp2, att_vec2):
  src = edge_index[0]
  dst = edge_index[1]

  w1 = jnp.stack([W_low1, W_high1, W_mlp1])
  stack1 = _stackmm(x, w1)
  sc1 = _spmm128(stack1, src, dst, adj_low_w)
  fea1 = _attention(sc1[:N], sc1[ACCN:ACCN + N], stack1,
                    att_low1.reshape(1, -1), att_high1.reshape(1, -1),
                    att_mlp1.reshape(1, -1), att_vec1, final_relu=True)

  wcat2 = jnp.concatenate([W_low2, W_high2], axis=1)
  table2, hm2 = _l2mm(fea1, wcat2, W_mlp2)
  sc2 = _spmm2(table2, src, dst, adj_low_w)
  fea2 = _attention2(sc2[:N], sc2[ACCN:ACCN + N], table2, hm2,
                     att_low2.reshape(1, -1), att_high2.reshape(1, -1),
                     att_mlp2.reshape(1, -1), att_vec2)
  return fea2
